# baseline (speedup 1.0000x reference)
import functools

import jax
import jax.numpy as jnp
from jax import lax
from jax.experimental import pallas as pl
from jax.experimental.pallas import tpu as pltpu
from jax.experimental.pallas import tpu_sc as plsc

D = 64
ROWS = 4096
COLS = 200
B = ROWS * COLS
V = 1_000_000
VP = V // 2
NC = 2
NS = 16
NW = NC * NS
SCALE = 8.0

LPW = B // NW
C = 128
NG = LPW // C
NGPAIR = NG // 2


def _lookup(xf, tbl):
    mesh = plsc.VectorSubcoreMesh(core_axis_name="c", subcore_axis_name="s")

    @functools.partial(
        pl.kernel,
        mesh=mesh,
        out_type=jax.ShapeDtypeStruct((B, D), jnp.float32),
        scratch_types=[
            pltpu.VMEM((2, C), jnp.int32),
            pltpu.VMEM((2, C), jnp.int32),
            pltpu.VMEM((2, C, 128), jnp.float32),
            pltpu.VMEM((2, C, D), jnp.float32),
            pltpu.SemaphoreType.DMA,
            pltpu.SemaphoreType.DMA,
            pltpu.SemaphoreType.DMA,
            pltpu.SemaphoreType.DMA,
        ],
        compiler_params=pltpu.CompilerParams(needs_layout_passes=False),
    )
    def k(idx_hbm, tbl_hbm, out_hbm, idx_v, q_v, rows_v, pk_v, g0, g1, s0, s1):
        wid = lax.axis_index("s") * NC + lax.axis_index("c")
        base = wid * LPW
        gsem = (g0, g1)
        ssem = (s0, s1)
        lanes = lax.iota(jnp.int32, 16)

        def fire_gather(g, bf):
            r0 = base + g * C
            pltpu.sync_copy(idx_hbm.at[pl.ds(r0, C)], idx_v.at[bf])
            for kk in range(C // 16):
                sl = pl.ds(kk * 16, 16)
                q_v[bf, sl] = lax.shift_right_logical(idx_v[bf, sl], 1)
            pltpu.async_copy(tbl_hbm.at[q_v.at[bf]], rows_v.at[bf], gsem[bf])

        def wait_gather(bf):
            pltpu.make_async_copy(
                tbl_hbm.at[pl.ds(0, C)], rows_v.at[bf], gsem[bf]
            ).wait()

        def repack(bf):
            rows2d = rows_v.at[bf]
            pk2d = pk_v.at[bf]
            for kk in range(C // 16):
                sl = pl.ds(kk * 16, 16)
                rvec = lanes + (kk * 16)
                off = (idx_v[bf, sl] & 1) * D
                for c8 in range(D // 8):
                    vals = []
                    for c1 in range(8):
                        cvec = lanes * 0 + (c8 * 8 + c1)
                        vals.append(
                            plsc.load_gather(rows2d, [rvec, off + cvec]) * SCALE
                        )
                    for c1 in range(8):
                        cvec = lanes * 0 + (c8 * 8 + c1)
                        plsc.store_scatter(pk2d, [rvec, cvec], vals[c1])

        def start_store(g, bf):
            r0 = base + g * C
            pltpu.async_copy(pk_v.at[bf], out_hbm.at[pl.ds(r0, C)], ssem[bf])

        def wait_store(bf):
            pltpu.make_async_copy(
                pk_v.at[bf], out_hbm.at[pl.ds(0, C)], ssem[bf]
            ).wait()

        fire_gather(0, 0)

        def step(t, carry):
            ge = 2 * t
            wait_gather(0)
            repack(0)

            @pl.when(t > 0)
            def _():
                wait_store(1)

            fire_gather(ge + 1, 1)
            start_store(ge, 0)
            wait_gather(1)
            repack(1)
            wait_store(0)

            @pl.when(t < NGPAIR - 1)
            def _():
                fire_gather(ge + 2, 0)

            start_store(ge + 1, 1)
            return carry

        lax.fori_loop(0, NGPAIR, step, 0)
        wait_store(1)

    return k(xf, tbl)


def kernel(x, lut):
    xf = x.reshape(B)
    tbl = lut.reshape(VP, 128)
    out = _lookup(xf, tbl)
    return out.reshape(ROWS, COLS, D)


# speedup vs baseline: 2.2254x; 2.2254x over previous
import functools

import jax
import jax.numpy as jnp
from jax import lax
from jax.experimental import pallas as pl
from jax.experimental.pallas import tpu as pltpu
from jax.experimental.pallas import tpu_sc as plsc

D = 64
ROWS = 4096
COLS = 200
NC = 2
NS = 16
NW = NC * NS
RPW = ROWS // NW
NPAIR = RPW // 2
SCALE = 8.0


def _gather_scaled(x, lut):
    mesh = plsc.VectorSubcoreMesh(core_axis_name="c", subcore_axis_name="s")

    @functools.partial(
        pl.kernel,
        mesh=mesh,
        out_type=jax.ShapeDtypeStruct((ROWS, COLS, 128), jnp.float32),
        scratch_types=[
            pltpu.VMEM((2, COLS), jnp.int32),
            pltpu.VMEM((2, COLS, D), jnp.float32),
            pltpu.VMEM((2, COLS, 128), jnp.float32),
            pltpu.SemaphoreType.DMA,
            pltpu.SemaphoreType.DMA,
            pltpu.SemaphoreType.DMA,
            pltpu.SemaphoreType.DMA,
        ],
        compiler_params=pltpu.CompilerParams(use_tc_tiling_on_sc=False),
    )
    def k(idx_hbm, table_hbm, out_hbm, idx_v, rows_v, stg_v, g0, g1, s0, s1):
        wid = lax.axis_index("s") * NC + lax.axis_index("c")
        base = wid * RPW
        gsem = (g0, g1)
        ssem = (s0, s1)

        def fire_gather(g, b):
            r0 = base + g
            pltpu.sync_copy(idx_hbm.at[r0, :], idx_v.at[b])
            pltpu.async_copy(
                table_hbm.at[idx_v.at[b, pl.ds(0, 128)]],
                rows_v.at[b, pl.ds(0, 128), :],
                gsem[b],
            )
            pltpu.async_copy(
                table_hbm.at[idx_v.at[b, pl.ds(128, COLS - 128)]],
                rows_v.at[b, pl.ds(128, COLS - 128), :],
                gsem[b],
            )

        def wait_gather(b):
            pltpu.make_async_copy(
                table_hbm.at[pl.ds(0, COLS)], rows_v.at[b], gsem[b]
            ).wait()

        def start_store(g, b):
            r0 = base + g
            pltpu.async_copy(stg_v.at[b], out_hbm.at[r0, :, :], ssem[b])

        def wait_store(b):
            pltpu.make_async_copy(
                stg_v.at[b], out_hbm.at[0, :, :], ssem[b]
            ).wait()

        def scale_spread(b):
            @plsc.parallel_loop(0, COLS, step=1, unroll=8)
            def _(r):
                for q in range(D // 16):
                    sl = pl.ds(q * 16, 16)
                    stg_v[b, r, sl] = rows_v[b, r, sl] * SCALE

        fire_gather(0, 0)

        def step(t, carry):
            ge = 2 * t
            wait_gather(0)
            scale_spread(0)

            @pl.when(t > 0)
            def _():
                wait_store(1)

            fire_gather(ge + 1, 1)
            start_store(ge, 0)
            wait_gather(1)
            scale_spread(1)
            wait_store(0)

            @pl.when(t < NPAIR - 1)
            def _():
                fire_gather(ge + 2, 0)

            start_store(ge + 1, 1)
            return carry

        lax.fori_loop(0, NPAIR, step, 0)
        wait_store(1)

    return k(x, lut)


def kernel(x, lut):
    return _gather_scaled(x, lut)[:, :, :D]


# speedup vs baseline: 2.2646x; 1.0176x over previous
import functools

import jax
import jax.numpy as jnp
from jax import lax
from jax.experimental import pallas as pl
from jax.experimental.pallas import tpu as pltpu
from jax.experimental.pallas import tpu_sc as plsc

D = 64
ROWS = 4096
COLS = 200
NC = 2
NS = 16
NW = NC * NS
RPW = ROWS // NW
NPAIR = RPW // 2
SCALE = 8.0


def _gather_scaled(x, lut):
    mesh = plsc.VectorSubcoreMesh(core_axis_name="c", subcore_axis_name="s")

    @functools.partial(
        pl.kernel,
        mesh=mesh,
        out_type=jax.ShapeDtypeStruct((ROWS, COLS, 128), jnp.float32),
        scratch_types=[
            pltpu.VMEM((2, COLS), jnp.int32),
            pltpu.VMEM((2, COLS, D), jnp.float32),
            pltpu.SemaphoreType.DMA,
            pltpu.SemaphoreType.DMA,
            pltpu.SemaphoreType.DMA,
            pltpu.SemaphoreType.DMA,
        ],
        compiler_params=pltpu.CompilerParams(use_tc_tiling_on_sc=False),
    )
    def k(idx_hbm, table_hbm, out_hbm, idx_v, rows_v, g0, g1, s0, s1):
        wid = lax.axis_index("s") * NC + lax.axis_index("c")
        base = wid * RPW
        gsem = (g0, g1)
        ssem = (s0, s1)

        def fire_gather(g, b):
            r0 = base + g
            pltpu.sync_copy(idx_hbm.at[r0, :], idx_v.at[b])
            pltpu.async_copy(
                table_hbm.at[idx_v.at[b, pl.ds(0, 128)]],
                rows_v.at[b, pl.ds(0, 128), :],
                gsem[b],
            )
            pltpu.async_copy(
                table_hbm.at[idx_v.at[b, pl.ds(128, COLS - 128)]],
                rows_v.at[b, pl.ds(128, COLS - 128), :],
                gsem[b],
            )

        def wait_gather(b):
            pltpu.make_async_copy(
                table_hbm.at[pl.ds(0, COLS)], rows_v.at[b], gsem[b]
            ).wait()

        def start_store(g, b):
            r0 = base + g
            pltpu.async_copy(
                rows_v.at[b], out_hbm.at[r0, :, pl.ds(0, D)], ssem[b]
            )

        def wait_store(b):
            pltpu.make_async_copy(
                rows_v.at[b], out_hbm.at[0, :, pl.ds(0, D)], ssem[b]
            ).wait()

        def scale_spread(b):
            @plsc.parallel_loop(0, COLS, step=1, unroll=8)
            def _(r):
                for q in range(D // 16):
                    sl = pl.ds(q * 16, 16)
                    rows_v[b, r, sl] = rows_v[b, r, sl] * SCALE

        fire_gather(0, 0)

        def step(t, carry):
            ge = 2 * t
            wait_gather(0)
            scale_spread(0)

            @pl.when(t > 0)
            def _():
                wait_store(1)

            fire_gather(ge + 1, 1)
            start_store(ge, 0)
            wait_gather(1)
            scale_spread(1)
            wait_store(0)

            @pl.when(t < NPAIR - 1)
            def _():
                fire_gather(ge + 2, 0)

            start_store(ge + 1, 1)
            return carry

        lax.fori_loop(0, NPAIR, step, 0)
        wait_store(1)

    return k(x, lut)


def kernel(x, lut):
    return _gather_scaled(x, lut)[:, :, :D]


# speedup vs baseline: 2.4231x; 1.0700x over previous
import functools

import jax
import jax.numpy as jnp
from jax import lax
from jax.experimental import pallas as pl
from jax.experimental.pallas import tpu as pltpu
from jax.experimental.pallas import tpu_sc as plsc

D = 64
ROWS = 4096
COLS = 200
NC = 2
NS = 16
NW = NC * NS
RPW = ROWS // NW
GR = 2
NPAIR = RPW // GR // 2
SCALE = 8.0


def _gather_scaled(x, lut):
    mesh = plsc.VectorSubcoreMesh(core_axis_name="c", subcore_axis_name="s")

    @functools.partial(
        pl.kernel,
        mesh=mesh,
        out_type=jax.ShapeDtypeStruct((ROWS, COLS, 128), jnp.float32),
        scratch_types=[
            pltpu.VMEM((2, GR, COLS), jnp.int32),
            pltpu.VMEM((2, GR, COLS, D), jnp.float32),
            pltpu.SemaphoreType.DMA,
            pltpu.SemaphoreType.DMA,
            pltpu.SemaphoreType.DMA,
            pltpu.SemaphoreType.DMA,
        ],
        compiler_params=pltpu.CompilerParams(use_tc_tiling_on_sc=False),
    )
    def k(idx_hbm, table_hbm, out_hbm, idx_v, rows_v, g0, g1, s0, s1):
        wid = lax.axis_index("s") * NC + lax.axis_index("c")
        base = wid * RPW
        gsem = (g0, g1)
        ssem = (s0, s1)

        def fire_gather(g, b):
            r0 = base + g * GR
            pltpu.sync_copy(idx_hbm.at[pl.ds(r0, GR), :], idx_v.at[b])
            for i in range(GR):
                pltpu.async_copy(
                    table_hbm.at[idx_v.at[b, i, pl.ds(0, 128)]],
                    rows_v.at[b, i, pl.ds(0, 128), :],
                    gsem[b],
                )
                pltpu.async_copy(
                    table_hbm.at[idx_v.at[b, i, pl.ds(128, COLS - 128)]],
                    rows_v.at[b, i, pl.ds(128, COLS - 128), :],
                    gsem[b],
                )

        def wait_gather(b):
            pltpu.make_async_copy(
                out_hbm.at[pl.ds(0, GR), :, pl.ds(0, D)], rows_v.at[b], gsem[b]
            ).wait()

        def start_store(g, b):
            r0 = base + g * GR
            pltpu.async_copy(
                rows_v.at[b], out_hbm.at[pl.ds(r0, GR), :, pl.ds(0, D)], ssem[b]
            )

        def wait_store(b):
            pltpu.make_async_copy(
                rows_v.at[b], out_hbm.at[pl.ds(0, GR), :, pl.ds(0, D)], ssem[b]
            ).wait()

        def scale_spread(b):
            for i in range(GR):
                @plsc.parallel_loop(0, COLS, step=1, unroll=8)
                def _(r):
                    for q in range(D // 16):
                        sl = pl.ds(q * 16, 16)
                        rows_v[b, i, r, sl] = rows_v[b, i, r, sl] * SCALE

        fire_gather(0, 0)

        def step(t, carry):
            ge = 2 * t
            wait_gather(0)
            scale_spread(0)

            @pl.when(t > 0)
            def _():
                wait_store(1)

            fire_gather(ge + 1, 1)
            start_store(ge, 0)
            wait_gather(1)
            scale_spread(1)
            wait_store(0)

            @pl.when(t < NPAIR - 1)
            def _():
                fire_gather(ge + 2, 0)

            start_store(ge + 1, 1)
            return carry

        lax.fori_loop(0, NPAIR, step, 0)
        wait_store(1)

    return k(x, lut)


def kernel(x, lut):
    return _gather_scaled(x, lut)[:, :, :D]
